# baseline (speedup 1.0000x reference)
import functools

import jax
import jax.numpy as jnp
from jax import lax
from jax.experimental import pallas as pl
from jax.experimental.pallas import tpu as pltpu
from jax.experimental.pallas import tpu_sc as plsc

_NUM_EXPERTS = 64
_HIDDEN = 1024
_TOP_K = 2
_BT = 4096
_NC, _NS, _L = 2, 16, 16


def _matmul_body(x_ref, w_ref, logits_ref):
    logits_ref[...] = lax.dot_general(
        x_ref[...], w_ref[...], (((1,), (1,)), ((), ())),
        preferred_element_type=jnp.float32,
    )


def _tc_logits(hidden_states, W):
    T, H = hidden_states.shape
    E = W.shape[0]
    return pl.pallas_call(
        _matmul_body,
        grid=(T // _BT,),
        in_specs=[
            pl.BlockSpec((_BT, H), lambda i: (i, 0)),
            pl.BlockSpec((E, H), lambda i: (0, 0)),
        ],
        out_specs=pl.BlockSpec((_BT, E), lambda i: (i, 0)),
        out_shape=jax.ShapeDtypeStruct((T, E), jnp.float32),
    )(hidden_states, W)


def _sc_topk(logits):
    T, E = logits.shape
    nw = _NC * _NS
    ntok = T // nw
    ngrp = ntok // _L

    mesh = plsc.VectorSubcoreMesh(core_axis_name="c", subcore_axis_name="s")

    @functools.partial(
        pl.kernel,
        out_type=[
            jax.ShapeDtypeStruct((T * _TOP_K,), jnp.float32),
            jax.ShapeDtypeStruct((T * _TOP_K,), jnp.int32),
        ],
        mesh=mesh,
        compiler_params=pltpu.CompilerParams(needs_layout_passes=False),
        scratch_types=[
            pltpu.VMEM((ntok * E,), jnp.float32),
            pltpu.VMEM((ntok * _TOP_K,), jnp.float32),
            pltpu.VMEM((ntok * _TOP_K,), jnp.int32),
        ],
    )
    def sc_kernel(logits_hbm, wout_hbm, iout_hbm, lg_v, wv, iv):
        wid = lax.axis_index("s") * _NC + lax.axis_index("c")
        base = wid * ntok
        pltpu.sync_copy(logits_hbm.at[pl.ds(base * E, ntok * E)], lg_v)

        lane = lax.iota(jnp.int32, _L)
        neg_inf = jnp.full((_L,), -jnp.inf, jnp.float32)
        zero_i = jnp.zeros((_L,), jnp.int32)
        U = 4

        def group(g, carry):
            tok = [(g * U + u) * _L + lane for u in range(U)]
            row0 = [t * E for t in tok]
            m1 = [neg_inf] * U
            m2 = [neg_inf] * U
            i1 = [zero_i] * U
            i2 = [zero_i] * U
            for e in range(E):
                col = jnp.full((_L,), e, jnp.int32)
                for u in range(U):
                    v = plsc.load_gather(lg_v, [row0[u] + e])
                    gt1 = v > m1[u]
                    gt2 = v > m2[u]
                    m2[u] = jnp.maximum(m2[u], jnp.minimum(m1[u], v))
                    m1[u] = jnp.maximum(m1[u], v)
                    i2[u] = jnp.where(gt1, i1[u],
                                      jnp.where(gt2, col, i2[u]))
                    i1[u] = jnp.where(gt1, col, i1[u])
            for u in range(U):
                e2 = jnp.exp(m2[u] - m1[u])
                w1 = 1.0 / (1.0 + e2)
                w2 = 1.0 - w1
                out0 = tok[u] * _TOP_K
                plsc.store_scatter(wv, [out0], w1)
                plsc.store_scatter(wv, [out0 + 1], w2)
                plsc.store_scatter(iv, [out0], i1[u])
                plsc.store_scatter(iv, [out0 + 1], i2[u])
            return carry

        lax.fori_loop(0, ngrp // U, group, 0)
        pltpu.sync_copy(wv, wout_hbm.at[pl.ds(base * _TOP_K, ntok * _TOP_K)])
        pltpu.sync_copy(iv, iout_hbm.at[pl.ds(base * _TOP_K, ntok * _TOP_K)])

    return sc_kernel(logits.reshape(T * E))


def kernel(hidden_states, W):
    logits = _tc_logits(hidden_states, W)
    wout, iout = _sc_topk(logits)
    T = hidden_states.shape[0]
    return (wout.reshape(T, _TOP_K), logits, iout.reshape(T, _TOP_K))


# speedup vs baseline: 1.0480x; 1.0480x over previous
import functools

import jax
import jax.numpy as jnp
from jax import lax
from jax.experimental import pallas as pl
from jax.experimental.pallas import tpu as pltpu
from jax.experimental.pallas import tpu_sc as plsc

_NUM_EXPERTS = 64
_HIDDEN = 1024
_TOP_K = 2
_BT = 4096
_NC, _NS, _L = 2, 16, 16


def _matmul_body(x_ref, w_ref, logits_ref):
    logits_ref[...] = lax.dot_general(
        x_ref[...], w_ref[...], (((1,), (1,)), ((), ())),
        preferred_element_type=jnp.float32,
    )


def _tc_logits_chunk(hidden_states, W, chunk, chunk_tokens):
    T, H = hidden_states.shape
    E = W.shape[0]
    bt = min(_BT, chunk_tokens)
    base_blk = chunk * (chunk_tokens // bt)
    return pl.pallas_call(
        _matmul_body,
        grid=(chunk_tokens // bt,),
        in_specs=[
            pl.BlockSpec((bt, H), lambda i: (base_blk + i, 0)),
            pl.BlockSpec((E, H), lambda i: (0, 0)),
        ],
        out_specs=pl.BlockSpec((bt, E), lambda i: (i, 0)),
        out_shape=jax.ShapeDtypeStruct((chunk_tokens, E), jnp.float32),
    )(hidden_states, W)


def _sc_topk(logits):
    T, E = logits.shape
    nw = _NC * _NS
    ntok = T // nw
    ngrp = ntok // _L

    mesh = plsc.VectorSubcoreMesh(core_axis_name="c", subcore_axis_name="s")

    @functools.partial(
        pl.kernel,
        out_type=[
            jax.ShapeDtypeStruct((T * _TOP_K,), jnp.float32),
            jax.ShapeDtypeStruct((T * _TOP_K,), jnp.int32),
        ],
        mesh=mesh,
        compiler_params=pltpu.CompilerParams(needs_layout_passes=False),
        scratch_types=[
            pltpu.VMEM((ntok * E,), jnp.float32),
            pltpu.VMEM((ntok * _TOP_K,), jnp.float32),
            pltpu.VMEM((ntok * _TOP_K,), jnp.int32),
        ],
    )
    def sc_kernel(logits_hbm, wout_hbm, iout_hbm, lg_v, wv, iv):
        wid = lax.axis_index("s") * _NC + lax.axis_index("c")
        base = wid * ntok
        pltpu.sync_copy(logits_hbm.at[pl.ds(base * E, ntok * E)], lg_v)

        lane = lax.iota(jnp.int32, _L)
        neg_inf = jnp.full((_L,), -jnp.inf, jnp.float32)
        zero_i = jnp.zeros((_L,), jnp.int32)
        U = 4

        def group(g, carry):
            tok = [(g * U + u) * _L + lane for u in range(U)]
            row0 = [t * E for t in tok]
            m1 = [neg_inf] * U
            m2 = [neg_inf] * U
            i1 = [zero_i] * U
            i2 = [zero_i] * U
            for e in range(E):
                col = jnp.full((_L,), e, jnp.int32)
                for u in range(U):
                    v = plsc.load_gather(lg_v, [row0[u] + e])
                    gt1 = v > m1[u]
                    gt2 = v > m2[u]
                    m2[u] = jnp.maximum(m2[u], jnp.minimum(m1[u], v))
                    m1[u] = jnp.maximum(m1[u], v)
                    i2[u] = jnp.where(gt1, i1[u],
                                      jnp.where(gt2, col, i2[u]))
                    i1[u] = jnp.where(gt1, col, i1[u])
            for u in range(U):
                e2 = jnp.exp(m2[u] - m1[u])
                w1 = 1.0 / (1.0 + e2)
                w2 = 1.0 - w1
                out0 = tok[u] * _TOP_K
                plsc.store_scatter(wv, [out0], w1)
                plsc.store_scatter(wv, [out0 + 1], w2)
                plsc.store_scatter(iv, [out0], i1[u])
                plsc.store_scatter(iv, [out0 + 1], i2[u])
            return carry

        lax.fori_loop(0, ngrp // U, group, 0)
        pltpu.sync_copy(wv, wout_hbm.at[pl.ds(base * _TOP_K, ntok * _TOP_K)])
        pltpu.sync_copy(iv, iout_hbm.at[pl.ds(base * _TOP_K, ntok * _TOP_K)])

    return sc_kernel(logits.reshape(T * E))


_CHUNKS = 4


def kernel(hidden_states, W):
    T = hidden_states.shape[0]
    ct = T // _CHUNKS
    lg_parts, w_parts, i_parts = [], [], []
    for c in range(_CHUNKS):
        lc = _tc_logits_chunk(hidden_states, W, c, ct)
        lg_parts.append(lc)
        wc, ic = _sc_topk(lc)
        w_parts.append(wc.reshape(ct, _TOP_K))
        i_parts.append(ic.reshape(ct, _TOP_K))
    return (
        jnp.concatenate(w_parts, axis=0),
        jnp.concatenate(lg_parts, axis=0),
        jnp.concatenate(i_parts, axis=0),
    )


# speedup vs baseline: 1.0498x; 1.0017x over previous
import functools

import jax
import jax.numpy as jnp
from jax import lax
from jax.experimental import pallas as pl
from jax.experimental.pallas import tpu as pltpu
from jax.experimental.pallas import tpu_sc as plsc

_NUM_EXPERTS = 64
_HIDDEN = 1024
_TOP_K = 2
_BT = 4096
_NC, _NS, _L = 2, 16, 16


def _matmul_body(x_ref, w_ref, logits_ref):
    logits_ref[...] = lax.dot_general(
        x_ref[...], w_ref[...], (((1,), (1,)), ((), ())),
        preferred_element_type=jnp.float32,
    )


def _tc_logits_chunk(hidden_states, W, chunk, chunk_tokens):
    T, H = hidden_states.shape
    E = W.shape[0]
    bt = min(_BT, chunk_tokens)
    base_blk = chunk * (chunk_tokens // bt)
    return pl.pallas_call(
        _matmul_body,
        grid=(chunk_tokens // bt,),
        in_specs=[
            pl.BlockSpec((bt, H), lambda i: (base_blk + i, 0)),
            pl.BlockSpec((E, H), lambda i: (0, 0)),
        ],
        out_specs=pl.BlockSpec((bt, E), lambda i: (i, 0)),
        out_shape=jax.ShapeDtypeStruct((chunk_tokens, E), jnp.float32),
    )(hidden_states, W)


def _sc_topk(logits):
    T, E = logits.shape
    nw = _NC * _NS
    ntok = T // nw
    ngrp = ntok // _L

    mesh = plsc.VectorSubcoreMesh(core_axis_name="c", subcore_axis_name="s")

    @functools.partial(
        pl.kernel,
        out_type=[
            jax.ShapeDtypeStruct((T * _TOP_K,), jnp.float32),
            jax.ShapeDtypeStruct((T * _TOP_K,), jnp.int32),
        ],
        mesh=mesh,
        compiler_params=pltpu.CompilerParams(needs_layout_passes=False),
        scratch_types=[
            pltpu.VMEM((ntok * E,), jnp.float32),
            pltpu.VMEM((ntok * _TOP_K,), jnp.float32),
            pltpu.VMEM((ntok * _TOP_K,), jnp.int32),
        ],
    )
    def sc_kernel(logits_hbm, wout_hbm, iout_hbm, lg_v, wv, iv):
        wid = lax.axis_index("s") * _NC + lax.axis_index("c")
        base = wid * ntok
        pltpu.sync_copy(logits_hbm.at[pl.ds(base * E, ntok * E)], lg_v)

        lane = lax.iota(jnp.int32, _L)
        neg_inf = jnp.full((_L,), -jnp.inf, jnp.float32)
        zero_i = jnp.zeros((_L,), jnp.int32)
        U = 2

        def group(g, carry):
            tok = [(g * U + u) * _L + lane for u in range(U)]
            row0 = [t * E for t in tok]
            m1 = [neg_inf] * U
            m2 = [neg_inf] * U
            i1 = [zero_i] * U
            i2 = [zero_i] * U
            for e in range(E):
                col = jnp.bitwise_and(lane + e, E - 1)
                for u in range(U):
                    v = plsc.load_gather(lg_v, [row0[u] + col])
                    gt1 = (v > m1[u]) | ((v == m1[u]) & (col < i1[u]))
                    gt2 = (v > m2[u]) | ((v == m2[u]) & (col < i2[u]))
                    m2[u] = jnp.where(gt1, m1[u], jnp.where(gt2, v, m2[u]))
                    i2[u] = jnp.where(gt1, i1[u],
                                      jnp.where(gt2, col, i2[u]))
                    m1[u] = jnp.where(gt1, v, m1[u])
                    i1[u] = jnp.where(gt1, col, i1[u])
            for u in range(U):
                e2 = jnp.exp(m2[u] - m1[u])
                w1 = 1.0 / (1.0 + e2)
                w2 = 1.0 - w1
                out0 = tok[u] * _TOP_K
                plsc.store_scatter(wv, [out0], w1)
                plsc.store_scatter(wv, [out0 + 1], w2)
                plsc.store_scatter(iv, [out0], i1[u])
                plsc.store_scatter(iv, [out0 + 1], i2[u])
            return carry

        lax.fori_loop(0, ngrp // U, group, 0)
        pltpu.sync_copy(wv, wout_hbm.at[pl.ds(base * _TOP_K, ntok * _TOP_K)])
        pltpu.sync_copy(iv, iout_hbm.at[pl.ds(base * _TOP_K, ntok * _TOP_K)])

    return sc_kernel(logits.reshape(T * E))


_CHUNKS = 4


def kernel(hidden_states, W):
    T = hidden_states.shape[0]
    ct = T // _CHUNKS
    lg_parts, w_parts, i_parts = [], [], []
    for c in range(_CHUNKS):
        lc = _tc_logits_chunk(hidden_states, W, c, ct)
        lg_parts.append(lc)
        wc, ic = _sc_topk(lc)
        w_parts.append(wc.reshape(ct, _TOP_K))
        i_parts.append(ic.reshape(ct, _TOP_K))
    return (
        jnp.concatenate(w_parts, axis=0),
        jnp.concatenate(lg_parts, axis=0),
        jnp.concatenate(i_parts, axis=0),
    )


# speedup vs baseline: 2.8951x; 2.7578x over previous
import jax
import jax.numpy as jnp
from jax import lax
from jax.experimental import pallas as pl

_NUM_EXPERTS = 64
_HIDDEN = 1024
_TOP_K = 2
_BT = 4096


def _router_body(x_ref, w_ref, wout_ref, logits_ref, iout_ref):
    step = pl.program_id(0)
    x = x_ref[...]
    w = w_ref[...]
    logits = lax.dot_general(
        x, w, (((1,), (1,)), ((), ())), preferred_element_type=jnp.float32
    )
    logits_ref[...] = logits

    lt = logits.T
    e_iota = lax.broadcasted_iota(jnp.int32, lt.shape, 0)
    m1 = jnp.max(lt, axis=0, keepdims=True)
    i1 = jnp.min(jnp.where(lt == m1, e_iota, _NUM_EXPERTS), axis=0,
                 keepdims=True)
    masked = jnp.where(e_iota == i1, -jnp.inf, lt)
    m2 = jnp.max(masked, axis=0, keepdims=True)
    i2 = jnp.min(jnp.where(masked == m2, e_iota, _NUM_EXPERTS), axis=0,
                 keepdims=True)

    e2 = jnp.exp(m2 - m1)
    s = 1.0 + e2
    w1 = 1.0 / s
    w2 = e2 / s
    cols = pl.ds(step * _BT, _BT)
    wout_ref[:, cols] = jnp.concatenate([w1, w2], axis=0)
    iout_ref[:, cols] = jnp.concatenate([i1, i2], axis=0)


def kernel(hidden_states, W):
    T, H = hidden_states.shape
    E = W.shape[0]
    grid = (T // _BT,)
    wout, logits, iout = pl.pallas_call(
        _router_body,
        grid=grid,
        in_specs=[
            pl.BlockSpec((_BT, H), lambda i: (i, 0)),
            pl.BlockSpec((E, H), lambda i: (0, 0)),
        ],
        out_specs=[
            pl.BlockSpec((_TOP_K, T), lambda i: (0, 0)),
            pl.BlockSpec((_BT, E), lambda i: (i, 0)),
            pl.BlockSpec((_TOP_K, T), lambda i: (0, 0)),
        ],
        out_shape=[
            jax.ShapeDtypeStruct((_TOP_K, T), jnp.float32),
            jax.ShapeDtypeStruct((T, E), jnp.float32),
            jax.ShapeDtypeStruct((_TOP_K, T), jnp.int32),
        ],
    )(hidden_states, W)
    return (wout.T, logits, iout.T)


# speedup vs baseline: 2.9248x; 1.0102x over previous
import jax
import jax.numpy as jnp
from jax import lax
from jax.experimental import pallas as pl

_NUM_EXPERTS = 64
_HIDDEN = 1024
_TOP_K = 2
_BT = 2048


def _router_body(x_ref, w_ref, wout_ref, logits_ref, iout_ref):
    step = pl.program_id(0)
    x = x_ref[...]
    w = w_ref[...]
    logits = lax.dot_general(
        x, w, (((1,), (1,)), ((), ())), preferred_element_type=jnp.float32
    )
    logits_ref[...] = logits

    lt = logits.T
    e_iota = lax.broadcasted_iota(jnp.int32, lt.shape, 0)
    m1 = jnp.max(lt, axis=0, keepdims=True)
    i1 = jnp.min(jnp.where(lt == m1, e_iota, _NUM_EXPERTS), axis=0,
                 keepdims=True)
    masked = jnp.where(e_iota == i1, -jnp.inf, lt)
    m2 = jnp.max(masked, axis=0, keepdims=True)
    i2 = jnp.min(jnp.where(masked == m2, e_iota, _NUM_EXPERTS), axis=0,
                 keepdims=True)

    e2 = jnp.exp(m2 - m1)
    s = 1.0 + e2
    w1 = 1.0 / s
    w2 = e2 / s
    cols = pl.ds(step * _BT, _BT)
    wout_ref[:, cols] = jnp.concatenate([w1, w2], axis=0)
    iout_ref[:, cols] = jnp.concatenate([i1, i2], axis=0)


def kernel(hidden_states, W):
    T, H = hidden_states.shape
    E = W.shape[0]
    grid = (T // _BT,)
    wout, logits, iout = pl.pallas_call(
        _router_body,
        grid=grid,
        in_specs=[
            pl.BlockSpec((_BT, H), lambda i: (i, 0)),
            pl.BlockSpec((E, H), lambda i: (0, 0)),
        ],
        out_specs=[
            pl.BlockSpec((_TOP_K, T), lambda i: (0, 0)),
            pl.BlockSpec((_BT, E), lambda i: (i, 0)),
            pl.BlockSpec((_TOP_K, T), lambda i: (0, 0)),
        ],
        out_shape=[
            jax.ShapeDtypeStruct((_TOP_K, T), jnp.float32),
            jax.ShapeDtypeStruct((T, E), jnp.float32),
            jax.ShapeDtypeStruct((_TOP_K, T), jnp.int32),
        ],
    )(hidden_states, W)
    return (wout.T, logits, iout.T)
